# baseline (speedup 1.0000x reference)
import functools

import jax
import jax.numpy as jnp
from jax import lax
from jax.experimental import pallas as pl
from jax.experimental.pallas import tpu as pltpu
from jax.experimental.pallas import tpu_sc as plsc

N = 10000
E = 320000
D = 128
DOUT = 512

NC = 2
NS = 16
NW = NC * NS
CH = 128
NCHUNK = E // CH
S0 = NCHUNK // NW
REM = NCHUNK - S0 * NW
NPAD = 10112
ZR = NPAD // NS


def _sc_aggregate(ei, x, zrows):
    mesh = plsc.VectorSubcoreMesh(
        core_axis_name="c", subcore_axis_name="s", num_cores=NC)

    @functools.partial(
        pl.kernel,
        out_type=jax.ShapeDtypeStruct((NC, NPAD, D), jnp.float32),
        mesh=mesh,
        scratch_types=[
            pltpu.VMEM_SHARED((NPAD, D), jnp.float32),
            [pltpu.VMEM((2, CH), jnp.int32)] * 3,
            [pltpu.VMEM((CH, D), jnp.float32)] * 3,
            [pltpu.SemaphoreType.DMA] * 3,
            [pltpu.SemaphoreType.DMA] * 3,
            [pltpu.SemaphoreType.DMA] * 3,
        ],
    )
    def k(ei_hbm, x_hbm, z_hbm, out_hbm, acc, ib, gb, si, sg, ss):
        c = lax.axis_index("c")
        s = lax.axis_index("s")
        wid = s * NC + c

        def idx_copies(off, bi, go):
            dsc = pltpu.make_async_copy(
                ei_hbm.at[:, pl.ds(off, CH)], ib[bi], si[bi])
            if go:
                dsc.start()
            else:
                dsc.wait()

        def start_idx(j, bi):
            idx_copies((wid + NW * j) * CH, bi, True)

        def wait_idx(j, bi):
            idx_copies((wid + NW * j) * CH, bi, False)

        def start_gather(bi):
            pltpu.async_copy(x_hbm.at[ib[bi].at[1]], gb[bi], sg[bi])

        def wait_gather(bi):
            pltpu.make_async_copy(x_hbm.at[ib[bi].at[1]], gb[bi], sg[bi]).wait()

        def start_scatter(bi):
            pltpu.async_copy(gb[bi], acc.at[ib[bi].at[0]], ss[bi], add=True)

        def wait_scatter(bi):
            pltpu.make_async_copy(gb[bi], acc.at[ib[bi].at[0]], ss[bi]).wait()

        start_idx(0, 0)
        start_idx(1, 1)
        XL = N - (NS - 1) * ZR

        @pl.when(c == 0)
        def _():
            @pl.when(s < NS - 1)
            def _():
                pltpu.sync_copy(x_hbm.at[pl.ds(s * ZR, ZR)],
                                acc.at[pl.ds(s * ZR, ZR)])

            @pl.when(s == NS - 1)
            def _():
                pltpu.sync_copy(x_hbm.at[pl.ds(s * ZR, XL)],
                                acc.at[pl.ds(s * ZR, XL)])
                pltpu.sync_copy(z_hbm.at[pl.ds(0, ZR - XL)],
                                acc.at[pl.ds(N, ZR - XL)])

        @pl.when(c == 1)
        def _():
            pltpu.sync_copy(z_hbm, acc.at[pl.ds(s * ZR, ZR)])

        plsc.subcore_barrier()
        wait_idx(0, 0)
        start_gather(0)

        def step(j, p, first=False, tail=0):
            wait_gather(p)
            start_scatter(p)
            if tail < 2:
                wait_idx(j + 1, (p + 1) % 3)
            if not first:
                wait_scatter((p + 2) % 3)
            if tail < 2:
                start_gather((p + 1) % 3)
            if tail < 1:
                start_idx(j + 2, (p + 2) % 3)

        step(0, 0, first=True)
        step(1, 1)
        step(2, 2)

        def body(t, carry):
            j = 3 * t
            step(j, 0)
            step(j + 1, 1)
            step(j + 2, 2)
            return carry

        lax.fori_loop(1, S0 // 3 - 1, body, 0)
        step(S0 - 3, 0)
        step(S0 - 2, 1, tail=1)
        step(S0 - 1, 2, tail=2)
        wait_scatter(2)

        @pl.when(wid < REM)
        def _():
            off = (S0 * NW + wid) * CH
            idx_copies(off, 0, True)
            idx_copies(off, 0, False)
            pltpu.async_copy(x_hbm.at[ib[0].at[1]], gb[0], sg[0]).wait()
            pltpu.sync_copy(gb[0], acc.at[ib[0].at[0]], add=True)

        plsc.subcore_barrier()
        pltpu.sync_copy(acc.at[pl.ds(s * ZR, ZR)],
                        out_hbm.at[c, pl.ds(s * ZR, ZR)])

    return k(ei, x, zrows)


def _linear(agg, W, b2):
    R = 2000
    grid = (N // R,)

    def mm(a_ref, w_ref, b_ref, o_ref):
        y = a_ref[0] + a_ref[1]
        o_ref[...] = lax.dot_general(
            y, w_ref[...], (((1,), (1,)), ((), ())),
            preferred_element_type=jnp.float32) + b_ref[...]

    return pl.pallas_call(
        mm,
        grid=grid,
        in_specs=[
            pl.BlockSpec((NC, R, D), lambda i: (0, i, 0)),
            pl.BlockSpec((DOUT, D), lambda i: (0, 0)),
            pl.BlockSpec((1, DOUT), lambda i: (0, 0)),
        ],
        out_specs=pl.BlockSpec((R, DOUT), lambda i: (i, 0)),
        out_shape=jax.ShapeDtypeStruct((N, DOUT), jnp.float32),
    )(agg, W, b2)


def kernel(x, edge_index, W, b):
    ei = edge_index.astype(jnp.int32)
    zrows = jnp.zeros((ZR, D), jnp.float32)
    agg = _sc_aggregate(ei, x, zrows)
    return _linear(agg, W, b[None, :])


# speedup vs baseline: 1.1576x; 1.1576x over previous
import functools

import jax
import jax.numpy as jnp
from jax import lax
from jax.experimental import pallas as pl
from jax.experimental.pallas import tpu as pltpu
from jax.experimental.pallas import tpu_sc as plsc

N = 10000
E = 320000
D = 128
DOUT = 512

NC = 2
NS = 16
NW = NC * NS
CH = 128
NCHUNK = E // CH
S0 = NCHUNK // NW
REM = NCHUNK - S0 * NW
NPAD = 10112
ZR = NPAD // NS


def _sc_aggregate(ei, x, zrows):
    mesh = plsc.VectorSubcoreMesh(
        core_axis_name="c", subcore_axis_name="s", num_cores=NC)

    @functools.partial(
        pl.kernel,
        out_type=jax.ShapeDtypeStruct((NC, NPAD, D), jnp.float32),
        mesh=mesh,
        scratch_types=[
            pltpu.VMEM_SHARED((NPAD, D), jnp.float32),
            [pltpu.VMEM((2, CH), jnp.int32)] * 3,
            [pltpu.VMEM((CH, D), jnp.float32)] * 3,
            [pltpu.SemaphoreType.DMA] * 3,
            [pltpu.SemaphoreType.DMA] * 3,
            [pltpu.SemaphoreType.DMA] * 3,
        ],
    )
    def k(ei_hbm, x_hbm, z_hbm, out_hbm, acc, ib, gb, si, sg, ss):
        c = lax.axis_index("c")
        s = lax.axis_index("s")
        wid = s * NC + c

        def idx_copies(off, bi, go):
            dsc = pltpu.make_async_copy(
                ei_hbm.at[:, pl.ds(off, CH)], ib[bi], si[bi])
            if go:
                dsc.start()
            else:
                dsc.wait()

        def start_idx(j, bi):
            idx_copies((wid + NW * j) * CH, bi, True)

        def wait_idx(j, bi):
            idx_copies((wid + NW * j) * CH, bi, False)

        def start_gather(bi):
            pltpu.async_copy(x_hbm.at[ib[bi].at[1]], gb[bi], sg[bi])

        def wait_gather(bi):
            pltpu.make_async_copy(x_hbm.at[ib[bi].at[1]], gb[bi], sg[bi]).wait()

        def start_scatter(bi):
            pltpu.async_copy(gb[bi], acc.at[ib[bi].at[0]], ss[bi], add=True)

        def wait_scatter(bi):
            pltpu.make_async_copy(gb[bi], acc.at[ib[bi].at[0]], ss[bi]).wait()

        start_idx(0, 0)
        start_idx(1, 1)
        XL = N - (NS - 1) * ZR

        @pl.when(c == 0)
        def _():
            @pl.when(s < NS - 1)
            def _():
                pltpu.sync_copy(x_hbm.at[pl.ds(s * ZR, ZR)],
                                acc.at[pl.ds(s * ZR, ZR)])

            @pl.when(s == NS - 1)
            def _():
                pltpu.sync_copy(x_hbm.at[pl.ds(s * ZR, XL)],
                                acc.at[pl.ds(s * ZR, XL)])
                pltpu.sync_copy(z_hbm.at[pl.ds(0, ZR - XL)],
                                acc.at[pl.ds(N, ZR - XL)])

        @pl.when(c == 1)
        def _():
            pltpu.sync_copy(z_hbm, acc.at[pl.ds(s * ZR, ZR)])

        plsc.subcore_barrier()
        wait_idx(0, 0)
        start_gather(0)

        def step(j, p, first=False, tail=0):
            if tail < 2:
                wait_idx(j + 1, (p + 1) % 3)
                start_gather((p + 1) % 3)
            wait_gather(p)
            start_scatter(p)
            if not first:
                wait_scatter((p + 2) % 3)
            if tail < 1:
                start_idx(j + 2, (p + 2) % 3)

        step(0, 0, first=True)
        step(1, 1)
        step(2, 2)

        def body(t, carry):
            j = 3 * t
            step(j, 0)
            step(j + 1, 1)
            step(j + 2, 2)
            return carry

        lax.fori_loop(1, S0 // 3 - 1, body, 0)
        step(S0 - 3, 0)
        step(S0 - 2, 1, tail=1)
        step(S0 - 1, 2, tail=2)
        wait_scatter(2)

        @pl.when(wid < REM)
        def _():
            off = (S0 * NW + wid) * CH
            idx_copies(off, 0, True)
            idx_copies(off, 0, False)
            pltpu.async_copy(x_hbm.at[ib[0].at[1]], gb[0], sg[0]).wait()
            pltpu.sync_copy(gb[0], acc.at[ib[0].at[0]], add=True)

        plsc.subcore_barrier()
        pltpu.sync_copy(acc.at[pl.ds(s * ZR, ZR)],
                        out_hbm.at[c, pl.ds(s * ZR, ZR)])

    return k(ei, x, zrows)


def _linear(agg, W, b2):
    R = 2000
    grid = (N // R,)

    def mm(a_ref, w_ref, b_ref, o_ref):
        y = a_ref[0] + a_ref[1]
        o_ref[...] = lax.dot_general(
            y, w_ref[...], (((1,), (1,)), ((), ())),
            preferred_element_type=jnp.float32) + b_ref[...]

    return pl.pallas_call(
        mm,
        grid=grid,
        in_specs=[
            pl.BlockSpec((NC, R, D), lambda i: (0, i, 0)),
            pl.BlockSpec((DOUT, D), lambda i: (0, 0)),
            pl.BlockSpec((1, DOUT), lambda i: (0, 0)),
        ],
        out_specs=pl.BlockSpec((R, DOUT), lambda i: (i, 0)),
        out_shape=jax.ShapeDtypeStruct((N, DOUT), jnp.float32),
    )(agg, W, b2)


def kernel(x, edge_index, W, b):
    ei = edge_index.astype(jnp.int32)
    zrows = jnp.zeros((ZR, D), jnp.float32)
    agg = _sc_aggregate(ei, x, zrows)
    return _linear(agg, W, b[None, :])


# speedup vs baseline: 1.1662x; 1.0074x over previous
import functools

import jax
import jax.numpy as jnp
from jax import lax
from jax.experimental import pallas as pl
from jax.experimental.pallas import tpu as pltpu
from jax.experimental.pallas import tpu_sc as plsc

N = 10000
E = 320000
D = 128
DOUT = 512

NC = 2
NS = 16
NW = NC * NS
CH = 128
NCHUNK = E // CH
S0 = NCHUNK // NW
REM = NCHUNK - S0 * NW
NPAD = 10112
ZR = NPAD // NS


def _sc_aggregate(ei, x, zrows):
    mesh = plsc.VectorSubcoreMesh(
        core_axis_name="c", subcore_axis_name="s", num_cores=NC)

    @functools.partial(
        pl.kernel,
        out_type=jax.ShapeDtypeStruct((NC, NPAD, D), jnp.float32),
        mesh=mesh,
        scratch_types=[
            pltpu.VMEM_SHARED((NPAD, D), jnp.float32),
            [pltpu.VMEM((2, CH), jnp.int32)] * 3,
            [pltpu.VMEM((CH, D), jnp.float32)] * 3,
            [pltpu.SemaphoreType.DMA] * 3,
            [pltpu.SemaphoreType.DMA] * 3,
            [pltpu.SemaphoreType.DMA] * 3,
        ],
    )
    def k(ei_hbm, x_hbm, z_hbm, out_hbm, acc, ib, gb, si, sg, ss):
        c = lax.axis_index("c")
        s = lax.axis_index("s")
        wid = s * NC + c

        def idx_copies(off, bi, go):
            dsc = pltpu.make_async_copy(
                ei_hbm.at[:, pl.ds(off, CH)], ib[bi], si[bi])
            if go:
                dsc.start()
            else:
                dsc.wait()

        def start_idx(j, bi):
            idx_copies((wid + NW * j) * CH, bi, True)

        def wait_idx(j, bi):
            idx_copies((wid + NW * j) * CH, bi, False)

        def start_gather(bi):
            pltpu.async_copy(x_hbm.at[ib[bi].at[1]], gb[bi], sg[bi])

        def wait_gather(bi):
            pltpu.make_async_copy(x_hbm.at[ib[bi].at[1]], gb[bi], sg[bi]).wait()

        def start_scatter(bi):
            pltpu.async_copy(gb[bi], acc.at[ib[bi].at[0]], ss[bi], add=True)

        def wait_scatter(bi):
            pltpu.make_async_copy(gb[bi], acc.at[ib[bi].at[0]], ss[bi]).wait()

        start_idx(0, 0)
        start_idx(1, 1)
        XL = N - (NS - 1) * ZR

        @pl.when(c == 0)
        def _():
            @pl.when(s < NS - 1)
            def _():
                pltpu.sync_copy(x_hbm.at[pl.ds(s * ZR, ZR)],
                                acc.at[pl.ds(s * ZR, ZR)])

            @pl.when(s == NS - 1)
            def _():
                pltpu.sync_copy(x_hbm.at[pl.ds(s * ZR, XL)],
                                acc.at[pl.ds(s * ZR, XL)])
                pltpu.sync_copy(z_hbm.at[pl.ds(0, ZR - XL)],
                                acc.at[pl.ds(N, ZR - XL)])

        @pl.when(c == 1)
        def _():
            pltpu.sync_copy(z_hbm, acc.at[pl.ds(s * ZR, ZR)])

        plsc.subcore_barrier()
        wait_idx(0, 0)
        start_gather(0)

        def step(j, p, first=False, tail=0):
            if tail < 2:
                wait_idx(j + 1, (p + 1) % 3)
            if not first:
                wait_scatter((p + 2) % 3)
            if tail < 2:
                start_gather((p + 1) % 3)
            if tail < 1:
                start_idx(j + 2, (p + 2) % 3)
            wait_gather(p)
            start_scatter(p)

        step(0, 0, first=True)
        step(1, 1)
        step(2, 2)

        def body(t, carry):
            j = 3 * t
            step(j, 0)
            step(j + 1, 1)
            step(j + 2, 2)
            return carry

        lax.fori_loop(1, S0 // 3 - 1, body, 0)
        step(S0 - 3, 0)
        step(S0 - 2, 1, tail=1)
        step(S0 - 1, 2, tail=2)
        wait_scatter(2)

        @pl.when(wid < REM)
        def _():
            off = (S0 * NW + wid) * CH
            idx_copies(off, 0, True)
            idx_copies(off, 0, False)
            pltpu.async_copy(x_hbm.at[ib[0].at[1]], gb[0], sg[0]).wait()
            pltpu.sync_copy(gb[0], acc.at[ib[0].at[0]], add=True)

        plsc.subcore_barrier()
        pltpu.sync_copy(acc.at[pl.ds(s * ZR, ZR)],
                        out_hbm.at[c, pl.ds(s * ZR, ZR)])

    return k(ei, x, zrows)


def _linear(agg, W, b2):
    R = 2000
    grid = (N // R,)

    def mm(a_ref, w_ref, b_ref, o_ref):
        y = a_ref[0] + a_ref[1]
        o_ref[...] = lax.dot_general(
            y, w_ref[...], (((1,), (1,)), ((), ())),
            preferred_element_type=jnp.float32) + b_ref[...]

    return pl.pallas_call(
        mm,
        grid=grid,
        in_specs=[
            pl.BlockSpec((NC, R, D), lambda i: (0, i, 0)),
            pl.BlockSpec((DOUT, D), lambda i: (0, 0)),
            pl.BlockSpec((1, DOUT), lambda i: (0, 0)),
        ],
        out_specs=pl.BlockSpec((R, DOUT), lambda i: (i, 0)),
        out_shape=jax.ShapeDtypeStruct((N, DOUT), jnp.float32),
    )(agg, W, b2)


def kernel(x, edge_index, W, b):
    ei = edge_index.astype(jnp.int32)
    zrows = jnp.zeros((ZR, D), jnp.float32)
    agg = _sc_aggregate(ei, x, zrows)
    return _linear(agg, W, b[None, :])
